# Initial kernel scaffold; baseline (speedup 1.0000x reference)
import jax
import jax.numpy as jnp
from jax.experimental import pallas as pl


def kernel(x, edge_index, W1, b1, W2, b2, W3, b3, Wfc, bfc):
    raise NotImplementedError("write your pallas kernel here")


# speedup vs baseline: 17.1002x; 17.1002x over previous
import functools

import jax
import jax.numpy as jnp
from jax import lax
from jax.experimental import pallas as pl
from jax.experimental.pallas import tpu as pltpu
from jax.experimental.pallas import tpu_sc as plsc

N = 10000
E = 320000
D = 128
DOUT = 64
NC, NS = 2, 16
NW = NC * NS
K = 80
EPW = E // NW
STEPS = EPW // K
ROWS_PT = 624
TAIL = N - NS * ROWS_PT
NPAD = 10240
DPT = NPAD // NS

def _deg_body(dst_hbm, deg_hbm, dacc, dst_v, ones_v, zero_v):
    c = lax.axis_index("c")
    s = lax.axis_index("s")
    wid = c * NS + s
    for k in range(K // 16):
        ones_v[pl.ds(k * 16, 16)] = jnp.ones((16,), jnp.float32)
    for k in range(DPT // 16):
        zero_v[pl.ds(k * 16, 16)] = jnp.zeros((16,), jnp.float32)
    pltpu.sync_copy(zero_v, dacc.at[pl.ds(s * DPT, DPT)])
    plsc.subcore_barrier()
    pltpu.sync_copy(dst_hbm.at[wid], dst_v)

    def body(j, _):
        pltpu.sync_copy(ones_v, dacc.at[dst_v.at[j]], add=True)
        return 0

    lax.fori_loop(0, STEPS, body, 0)
    plsc.subcore_barrier()
    pltpu.sync_copy(dacc.at[pl.ds(s * DPT, DPT)],
                    deg_hbm.at[c, pl.ds(s * DPT, DPT)])


def _prop_body(g_hbm, src_hbm, dst_hbm, zeros_hbm, out_hbm,
               acc, src_v, dst_v, rows_v, sem):
    c = lax.axis_index("c")
    s = lax.axis_index("s")
    wid = c * NS + s
    @pl.when(c == 0)
    def _():
        pltpu.sync_copy(g_hbm.at[pl.ds(s * ROWS_PT, ROWS_PT)],
                        acc.at[pl.ds(s * ROWS_PT, ROWS_PT)])

    @pl.when(c != 0)
    def _():
        pltpu.sync_copy(zeros_hbm.at[pl.ds(s * ROWS_PT, ROWS_PT)],
                        acc.at[pl.ds(s * ROWS_PT, ROWS_PT)])

    @pl.when((c == 0) & (s == 0))
    def _():
        pltpu.sync_copy(g_hbm.at[pl.ds(NS * ROWS_PT, TAIL)],
                        acc.at[pl.ds(NS * ROWS_PT, TAIL)])

    @pl.when((c != 0) & (s == 0))
    def _():
        pltpu.sync_copy(zeros_hbm.at[pl.ds(NS * ROWS_PT, TAIL)],
                        acc.at[pl.ds(NS * ROWS_PT, TAIL)])

    pltpu.sync_copy(src_hbm.at[wid], src_v)
    pltpu.sync_copy(dst_hbm.at[wid], dst_v)
    plsc.subcore_barrier()

    def body(j, _):
        pltpu.async_copy(g_hbm.at[src_v.at[j]], rows_v, sem).wait()
        pltpu.sync_copy(rows_v, acc.at[dst_v.at[j]], add=True)
        return 0

    lax.fori_loop(0, STEPS, body, 0)
    plsc.subcore_barrier()
    pltpu.sync_copy(acc.at[pl.ds(s * ROWS_PT, ROWS_PT)],
                    out_hbm.at[c, pl.ds(s * ROWS_PT, ROWS_PT)])

    @pl.when(s == 0)
    def _():
        pltpu.sync_copy(acc.at[pl.ds(NS * ROWS_PT, TAIL)],
                        out_hbm.at[c, pl.ds(NS * ROWS_PT, TAIL)])


@functools.lru_cache(maxsize=None)
def _sc_kernels():
    mesh = plsc.VectorSubcoreMesh(core_axis_name="c", subcore_axis_name="s")
    deg_k = pl.kernel(
        _deg_body,
        out_type=jax.ShapeDtypeStruct((NC, NPAD), jnp.float32),
        mesh=mesh,
        scratch_types=[
            pltpu.VMEM_SHARED((NPAD,), jnp.float32),
            pltpu.VMEM((STEPS, K), jnp.int32),
            pltpu.VMEM((K,), jnp.float32),
            pltpu.VMEM((DPT,), jnp.float32),
        ],
    )
    prop_k = pl.kernel(
        _prop_body,
        out_type=jax.ShapeDtypeStruct((NC, N, D), jnp.float32),
        mesh=mesh,
        scratch_types=[
            pltpu.VMEM_SHARED((N, D), jnp.float32),
            pltpu.VMEM((STEPS, K), jnp.int32),
            pltpu.VMEM((STEPS, K), jnp.int32),
            pltpu.VMEM((K, D), jnp.float32),
            pltpu.SemaphoreType.DMA,
        ],
    )
    return deg_k, prop_k


_BS = 2000


def _tc_prep_body(deg_ref, x_ref, w_ref, g_ref):
    d = deg_ref[...]
    dinv = lax.rsqrt(d[:, 0:1] + d[:, 1:2] + 1.0)
    g_ref[...] = dinv * jnp.dot(x_ref[...], w_ref[...],
                                preferred_element_type=jnp.float32)


def _tc_layer_body(deg_ref, s_ref, b_ref, w_ref, g_ref):
    d = deg_ref[...]
    dinv = lax.rsqrt(d[:, 0:1] + d[:, 1:2] + 1.0)
    h = jax.nn.relu(dinv * (s_ref[0] + s_ref[1]) + b_ref[...])
    g_ref[...] = dinv * jnp.dot(h, w_ref[...],
                                preferred_element_type=jnp.float32)


def _tc_final_body(deg_ref, s_ref, b_ref, wfc_ref, bfc_ref, out_ref, acc):
    i = pl.program_id(0)

    @pl.when(i == 0)
    def _():
        acc[...] = jnp.zeros_like(acc)

    d = deg_ref[...]
    dinv = lax.rsqrt(d[:, 0:1] + d[:, 1:2] + 1.0)
    h = jax.nn.relu(dinv * (s_ref[0] + s_ref[1]) + b_ref[...])
    acc[...] += jnp.sum(h, axis=0, keepdims=True)

    @pl.when(i == pl.num_programs(0) - 1)
    def _():
        pooled = acc[...] * (1.0 / N)
        out_ref[...] = jnp.dot(pooled, wfc_ref[...],
                               preferred_element_type=jnp.float32) + bfc_ref[...]


def _tc_prep(deg2, x, w):
    grid = N // _BS
    return pl.pallas_call(
        _tc_prep_body,
        grid=(grid,),
        in_specs=[
            pl.BlockSpec((_BS, 2), lambda i: (i, 0)),
            pl.BlockSpec((_BS, D), lambda i: (i, 0)),
            pl.BlockSpec((D, D), lambda i: (0, 0)),
        ],
        out_specs=pl.BlockSpec((_BS, D), lambda i: (i, 0)),
        out_shape=jax.ShapeDtypeStruct((N, D), jnp.float32),
    )(deg2, x, w)


def _tc_layer(deg2, s, b, w):
    grid = N // _BS
    return pl.pallas_call(
        _tc_layer_body,
        grid=(grid,),
        in_specs=[
            pl.BlockSpec((_BS, 2), lambda i: (i, 0)),
            pl.BlockSpec((NC, _BS, D), lambda i: (0, i, 0)),
            pl.BlockSpec((1, D), lambda i: (0, 0)),
            pl.BlockSpec((D, D), lambda i: (0, 0)),
        ],
        out_specs=pl.BlockSpec((_BS, D), lambda i: (i, 0)),
        out_shape=jax.ShapeDtypeStruct((N, D), jnp.float32),
    )(deg2, s, b, w)


def _tc_final(deg2, s, b, wfc, bfc):
    grid = N // _BS
    return pl.pallas_call(
        _tc_final_body,
        grid=(grid,),
        in_specs=[
            pl.BlockSpec((_BS, 2), lambda i: (i, 0)),
            pl.BlockSpec((NC, _BS, D), lambda i: (0, i, 0)),
            pl.BlockSpec((1, D), lambda i: (0, 0)),
            pl.BlockSpec((D, DOUT), lambda i: (0, 0)),
            pl.BlockSpec((1, DOUT), lambda i: (0, 0)),
        ],
        out_specs=pl.BlockSpec((1, DOUT), lambda i: (0, 0)),
        out_shape=jax.ShapeDtypeStruct((1, DOUT), jnp.float32),
        scratch_shapes=[pltpu.VMEM((1, D), jnp.float32)],
    )(deg2, s, b, wfc, bfc)


def kernel(x, edge_index, W1, b1, W2, b2, W3, b3, Wfc, bfc):
    src = edge_index[0].reshape(NW, STEPS, K)
    dst = edge_index[1].reshape(NW, STEPS, K)
    zeros = jnp.zeros((N, D), jnp.float32)

    deg_kernel, prop_kernel = _sc_kernels()
    degp = deg_kernel(dst)
    deg2 = jnp.transpose(degp)[:N]

    g = _tc_prep(deg2, x, W1)
    s = prop_kernel(g, src, dst, zeros)
    g = _tc_layer(deg2, s, b1.reshape(1, D), W2)
    s = prop_kernel(g, src, dst, zeros)
    g = _tc_layer(deg2, s, b2.reshape(1, D), W3)
    s = prop_kernel(g, src, dst, zeros)
    return _tc_final(deg2, s, b3.reshape(1, D), Wfc, bfc.reshape(1, DOUT))


# speedup vs baseline: 31.1538x; 1.8218x over previous
import functools

import jax
import jax.numpy as jnp
from jax import lax
from jax.experimental import pallas as pl
from jax.experimental.pallas import tpu as pltpu
from jax.experimental.pallas import tpu_sc as plsc

N = 10000
E = 320000
D = 128
DOUT = 64
NC, NS = 2, 16
NW = NC * NS
K = 80
EPW = E // NW
STEPS = EPW // K
NBUF = 3
ROWS_PT = 624
TAIL = N - NS * ROWS_PT
NPAD = 10240
DPT = NPAD // NS

def _deg_body(dst_hbm, deg_hbm, dacc, dst_v, ones_v, zero_v):
    c = lax.axis_index("c")
    s = lax.axis_index("s")
    wid = c * NS + s
    for k in range(K // 16):
        ones_v[pl.ds(k * 16, 16)] = jnp.ones((16,), jnp.float32)
    for k in range(DPT // 16):
        zero_v[pl.ds(k * 16, 16)] = jnp.zeros((16,), jnp.float32)
    pltpu.sync_copy(zero_v, dacc.at[pl.ds(s * DPT, DPT)])
    plsc.subcore_barrier()
    pltpu.sync_copy(dst_hbm.at[wid], dst_v)

    def body(j, _):
        pltpu.sync_copy(ones_v, dacc.at[dst_v.at[j]], add=True)
        return 0

    lax.fori_loop(0, STEPS, body, 0)
    plsc.subcore_barrier()
    pltpu.sync_copy(dacc.at[pl.ds(s * DPT, DPT)],
                    deg_hbm.at[c, pl.ds(s * DPT, DPT)])


def _prop_body(g_hbm, src_hbm, dst_hbm, zeros_hbm, out_hbm,
               acc, src_v, dst_v, dchunk, rows_v, sem):
    c = lax.axis_index("c")
    s = lax.axis_index("s")
    wid = c * NS + s
    @pl.when(c == 0)
    def _():
        pltpu.sync_copy(g_hbm.at[pl.ds(s * ROWS_PT, ROWS_PT)],
                        acc.at[pl.ds(s * ROWS_PT, ROWS_PT)])

    @pl.when(c != 0)
    def _():
        pltpu.sync_copy(zeros_hbm.at[pl.ds(s * ROWS_PT, ROWS_PT)],
                        acc.at[pl.ds(s * ROWS_PT, ROWS_PT)])

    @pl.when((c == 0) & (s == 0))
    def _():
        pltpu.sync_copy(g_hbm.at[pl.ds(NS * ROWS_PT, TAIL)],
                        acc.at[pl.ds(NS * ROWS_PT, TAIL)])

    @pl.when((c != 0) & (s == 0))
    def _():
        pltpu.sync_copy(zeros_hbm.at[pl.ds(NS * ROWS_PT, TAIL)],
                        acc.at[pl.ds(NS * ROWS_PT, TAIL)])

    pltpu.sync_copy(src_hbm.at[pl.ds(wid * EPW, EPW)], src_v)
    pltpu.sync_copy(dst_hbm.at[pl.ds(wid * EPW, EPW)], dst_v)
    plsc.subcore_barrier()

    def gidx(j):
        return src_v.at[pl.ds(j * K, K)]

    def start_gather(j, b):
        pltpu.async_copy(g_hbm.at[gidx(j)], rows_v.at[b], sem[b])

    def drain(j, b):
        pltpu.make_async_copy(g_hbm.at[gidx(j)], rows_v.at[b], sem[b]).wait()
        for k in range(K // 16):
            dchunk[pl.ds(k * 16, 16)] = dst_v[pl.ds(j * K + k * 16, 16)]
        pltpu.sync_copy(rows_v.at[b], acc.at[dchunk], add=True)

    MAIN = (STEPS - NBUF) // NBUF * NBUF
    for b in range(NBUF):
        start_gather(b, b)

    def outer(o, _):
        for b in range(NBUF):
            j = o * NBUF + b
            drain(j, b)
            start_gather(j + NBUF, b)
        return 0

    lax.fori_loop(0, MAIN // NBUF, outer, 0)
    for b in range(NBUF):
        drain(MAIN + b, b)
    for j in range(MAIN + NBUF, STEPS):
        start_gather(j, 0)
        drain(j, 0)
    plsc.subcore_barrier()
    pltpu.sync_copy(acc.at[pl.ds(s * ROWS_PT, ROWS_PT)],
                    out_hbm.at[c, pl.ds(s * ROWS_PT, ROWS_PT)])

    @pl.when(s == 0)
    def _():
        pltpu.sync_copy(acc.at[pl.ds(NS * ROWS_PT, TAIL)],
                        out_hbm.at[c, pl.ds(NS * ROWS_PT, TAIL)])


@functools.lru_cache(maxsize=None)
def _sc_kernels():
    mesh = plsc.VectorSubcoreMesh(core_axis_name="c", subcore_axis_name="s")
    deg_k = pl.kernel(
        _deg_body,
        out_type=jax.ShapeDtypeStruct((NC, NPAD), jnp.float32),
        mesh=mesh,
        scratch_types=[
            pltpu.VMEM_SHARED((NPAD,), jnp.float32),
            pltpu.VMEM((STEPS, K), jnp.int32),
            pltpu.VMEM((K,), jnp.float32),
            pltpu.VMEM((DPT,), jnp.float32),
        ],
    )
    prop_k = pl.kernel(
        _prop_body,
        out_type=jax.ShapeDtypeStruct((NC, N, D), jnp.float32),
        mesh=mesh,
        scratch_types=[
            pltpu.VMEM_SHARED((N, D), jnp.float32),
            pltpu.VMEM((EPW,), jnp.int32),
            pltpu.VMEM((EPW,), jnp.int32),
            pltpu.VMEM((K,), jnp.int32),
            pltpu.VMEM((NBUF, K, D), jnp.float32),
            [pltpu.SemaphoreType.DMA] * NBUF,
        ],
    )
    return deg_k, prop_k


_BS = 2000


def _tc_prep_body(deg_ref, x_ref, w_ref, g_ref):
    d = deg_ref[...]
    dinv = lax.rsqrt(d[:, 0:1] + d[:, 1:2] + 1.0)
    g_ref[...] = dinv * jnp.dot(x_ref[...], w_ref[...],
                                preferred_element_type=jnp.float32)


def _tc_layer_body(deg_ref, s_ref, b_ref, w_ref, g_ref):
    d = deg_ref[...]
    dinv = lax.rsqrt(d[:, 0:1] + d[:, 1:2] + 1.0)
    h = jax.nn.relu(dinv * (s_ref[0] + s_ref[1]) + b_ref[...])
    g_ref[...] = dinv * jnp.dot(h, w_ref[...],
                                preferred_element_type=jnp.float32)


def _tc_final_body(deg_ref, s_ref, b_ref, wfc_ref, bfc_ref, out_ref, acc):
    i = pl.program_id(0)

    @pl.when(i == 0)
    def _():
        acc[...] = jnp.zeros_like(acc)

    d = deg_ref[...]
    dinv = lax.rsqrt(d[:, 0:1] + d[:, 1:2] + 1.0)
    h = jax.nn.relu(dinv * (s_ref[0] + s_ref[1]) + b_ref[...])
    acc[...] += jnp.sum(h, axis=0, keepdims=True)

    @pl.when(i == pl.num_programs(0) - 1)
    def _():
        pooled = acc[...] * (1.0 / N)
        out_ref[...] = jnp.dot(pooled, wfc_ref[...],
                               preferred_element_type=jnp.float32) + bfc_ref[...]


def _tc_prep(deg2, x, w):
    grid = N // _BS
    return pl.pallas_call(
        _tc_prep_body,
        grid=(grid,),
        in_specs=[
            pl.BlockSpec((_BS, 2), lambda i: (i, 0)),
            pl.BlockSpec((_BS, D), lambda i: (i, 0)),
            pl.BlockSpec((D, D), lambda i: (0, 0)),
        ],
        out_specs=pl.BlockSpec((_BS, D), lambda i: (i, 0)),
        out_shape=jax.ShapeDtypeStruct((N, D), jnp.float32),
    )(deg2, x, w)


def _tc_layer(deg2, s, b, w):
    grid = N // _BS
    return pl.pallas_call(
        _tc_layer_body,
        grid=(grid,),
        in_specs=[
            pl.BlockSpec((_BS, 2), lambda i: (i, 0)),
            pl.BlockSpec((NC, _BS, D), lambda i: (0, i, 0)),
            pl.BlockSpec((1, D), lambda i: (0, 0)),
            pl.BlockSpec((D, D), lambda i: (0, 0)),
        ],
        out_specs=pl.BlockSpec((_BS, D), lambda i: (i, 0)),
        out_shape=jax.ShapeDtypeStruct((N, D), jnp.float32),
    )(deg2, s, b, w)


def _tc_final(deg2, s, b, wfc, bfc):
    grid = N // _BS
    return pl.pallas_call(
        _tc_final_body,
        grid=(grid,),
        in_specs=[
            pl.BlockSpec((_BS, 2), lambda i: (i, 0)),
            pl.BlockSpec((NC, _BS, D), lambda i: (0, i, 0)),
            pl.BlockSpec((1, D), lambda i: (0, 0)),
            pl.BlockSpec((D, DOUT), lambda i: (0, 0)),
            pl.BlockSpec((1, DOUT), lambda i: (0, 0)),
        ],
        out_specs=pl.BlockSpec((1, DOUT), lambda i: (0, 0)),
        out_shape=jax.ShapeDtypeStruct((1, DOUT), jnp.float32),
        scratch_shapes=[pltpu.VMEM((1, D), jnp.float32)],
    )(deg2, s, b, wfc, bfc)


def kernel(x, edge_index, W1, b1, W2, b2, W3, b3, Wfc, bfc):
    src = edge_index[0]
    dst = edge_index[1]
    dst3 = dst.reshape(NW, STEPS, K)
    zeros = jnp.zeros((N, D), jnp.float32)

    deg_kernel, prop_kernel = _sc_kernels()
    degp = deg_kernel(dst3)
    deg2 = jnp.transpose(degp)[:N]

    g = _tc_prep(deg2, x, W1)
    s = prop_kernel(g, src, dst, zeros)
    g = _tc_layer(deg2, s, b1.reshape(1, D), W2)
    s = prop_kernel(g, src, dst, zeros)
    g = _tc_layer(deg2, s, b2.reshape(1, D), W3)
    s = prop_kernel(g, src, dst, zeros)
    return _tc_final(deg2, s, b3.reshape(1, D), Wfc, bfc.reshape(1, DOUT))
